# baseline (speedup 1.0000x reference)
import jax
import jax.numpy as jnp
from jax.experimental import pallas as pl
from jax.experimental.pallas import tpu as pltpu

_NEG_POS_RATIO = 3
_INTERPRET = False


def _main_body(conf_ref, labels_ref, pred_ref, gt_ref, parts_ref,
               m_s, s_s, xsel_s, x0_s):
    nc = pl.num_programs(1)
    kc = pl.program_id(1)
    x = conf_ref[...]
    cc = x.shape[0]
    n = x.shape[2]
    lab = labels_ref[...]
    pos = lab > 0
    mx = jnp.max(x, axis=0)
    iota = jax.lax.broadcasted_iota(jnp.int32, x.shape, 0) + kc * cc
    xsel_c = jnp.sum(jnp.where(iota == lab[None], x, 0.0), axis=0)

    @pl.when(kc == 0)
    def _init():
        m_s[...] = mx
        s_s[...] = jnp.sum(jnp.exp(x - mx[None]), axis=0)
        xsel_s[...] = xsel_c
        x0_s[...] = x[0]
        d = pred_ref[...] - gt_ref[...]
        ad = jnp.abs(d)
        sl1 = jnp.where(ad < 1.0, 0.5 * d * d, ad - 0.5)
        masked = jnp.where(pos, jnp.sum(sl1, axis=1), 0.0)
        parts_ref[:, 1:2] = jnp.sum(masked, axis=1, keepdims=True)

    @pl.when(kc > 0)
    def _update():
        m_old = m_s[...]
        m_new = jnp.maximum(m_old, mx)
        s_s[...] = (s_s[...] * jnp.exp(m_old - m_new)
                    + jnp.sum(jnp.exp(x - m_new[None]), axis=0))
        m_s[...] = m_new
        xsel_s[...] = xsel_s[...] + xsel_c

    @pl.when(kc == nc - 1)
    def _emit():
        lse = m_s[...] + jnp.log(s_s[...])
        cep = jnp.where(pos, lse - xsel_s[...], 0.0)
        parts_ref[:, 0:1] = jnp.sum(cep, axis=1, keepdims=True)
        bg = lse - x0_s[...]
        num_pos = jnp.sum(pos.astype(jnp.int32), axis=1, keepdims=True)
        k = num_pos * _NEG_POS_RATIO
        negcount = n - num_pos
        bits = jax.lax.bitcast_convert_type(bg, jnp.uint32)
        key = jnp.where(bg >= 0, bits | jnp.uint32(0x80000000), ~bits)
        key = jnp.where(pos, jnp.uint32(0), key)
        sum_neg = jnp.sum(jnp.where(pos, 0.0, bg), axis=1, keepdims=True)
        prefix = jnp.zeros_like(num_pos, dtype=jnp.uint32)
        kk = k
        for bit in range(31, -1, -1):
            cand = prefix | jnp.uint32(1 << bit)
            match = (key >> jnp.uint32(bit)) == (cand >> jnp.uint32(bit))
            cnt = jnp.sum(match.astype(jnp.int32), axis=1, keepdims=True)
            take = cnt >= kk
            prefix = jnp.where(take, cand, prefix)
            kk = jnp.where(take, kk, kk - cnt)
        t = prefix
        gt_mask = key > t
        num_gt = jnp.sum(gt_mask.astype(jnp.int32), axis=1, keepdims=True)
        sum_gt = jnp.sum(jnp.where(gt_mask, bg, 0.0), axis=1, keepdims=True)
        vt = jnp.max(jnp.where(key == t, bg, -jnp.inf), axis=1, keepdims=True)
        topk = sum_gt + (k - num_gt).astype(jnp.float32) * vt
        topk = jnp.where(k >= negcount, sum_neg, topk)
        topk = jnp.where(k <= 0, 0.0, topk)
        parts_ref[:, 2:3] = topk
        parts_ref[:, 3:4] = num_pos.astype(jnp.float32)


def _combine_body(parts_ref, o1_ref, o2_ref):
    p = parts_ref[...]
    tot = jnp.sum(p, axis=0, keepdims=True)
    npos = tot[:, 3:4] + 1e-6
    o1_ref[...] = tot[:, 1:2] / npos
    o2_ref[...] = (tot[:, 0:1] + tot[:, 2:3]) / npos


def kernel(confidence, predicted_locations, labels, gt_locations):
    b, n, c = confidence.shape
    labels = labels.astype(jnp.int32)
    conf_t = confidence.transpose(2, 0, 1)
    pred_t = predicted_locations.transpose(0, 2, 1)
    gt_t = gt_locations.transpose(0, 2, 1)
    bb = 8
    cc = 9
    parts = pl.pallas_call(
        _main_body,
        grid=(b // bb, c // cc),
        in_specs=[
            pl.BlockSpec((cc, bb, n), lambda j, kc: (kc, j, 0)),
            pl.BlockSpec((bb, n), lambda j, kc: (j, 0)),
            pl.BlockSpec((bb, 4, n), lambda j, kc: (j, 0, 0)),
            pl.BlockSpec((bb, 4, n), lambda j, kc: (j, 0, 0)),
        ],
        out_specs=pl.BlockSpec((bb, 4), lambda j, kc: (j, 0)),
        out_shape=jax.ShapeDtypeStruct((b, 4), jnp.float32),
        scratch_shapes=[pltpu.VMEM((bb, n), jnp.float32) for _ in range(4)],
        interpret=_INTERPRET,
    )(conf_t, labels, pred_t, gt_t)

    o1, o2 = pl.pallas_call(
        _combine_body,
        out_shape=[
            jax.ShapeDtypeStruct((1, 1), jnp.float32),
            jax.ShapeDtypeStruct((1, 1), jnp.float32),
        ],
        interpret=_INTERPRET,
    )(parts)
    return (o1[0, 0], o2[0, 0])


# speedup vs baseline: 1.2356x; 1.2356x over previous
import functools

import jax
import jax.numpy as jnp
from jax.experimental import pallas as pl
from jax.experimental.pallas import tpu as pltpu

_NEG_POS_RATIO = 3
_INTERPRET = False


def _lse_body(cc, conf_ref, labels_ref, bg_ref, cep_ref, m_s, s_s, xsel_s, x0_s):
    nc = pl.num_programs(1)
    kc = pl.program_id(1)
    lab = labels_ref[...]
    base = kc * cc
    mx = conf_ref[0]
    sel = jnp.where(lab == base, mx, 0.0)
    for ci in range(1, cc):
        xi = conf_ref[ci]
        mx = jnp.maximum(mx, xi)
        sel = sel + jnp.where(lab == base + ci, xi, 0.0)
    first = kc == 0
    m_old = jnp.where(first, mx, m_s[...])
    m_new = jnp.maximum(m_old, mx)
    s = jnp.where(first, 0.0, s_s[...]) * jnp.exp(m_old - m_new)
    for ci in range(cc):
        s = s + jnp.exp(conf_ref[ci] - m_new)
    m_s[...] = m_new
    s_s[...] = s
    xsel_s[...] = jnp.where(first, sel, xsel_s[...] + sel)

    @pl.when(first)
    def _save0():
        x0_s[...] = conf_ref[0]

    @pl.when(kc == nc - 1)
    def _emit():
        lse = m_new + jnp.log(s)
        bg_ref[...] = lse - x0_s[...]
        cep = jnp.where(lab > 0, lse - xsel_s[...], 0.0)
        cep_ref[...] = jnp.sum(cep, axis=1, keepdims=True)


def _sl1_body(pred_ref, gt_ref, labels_ref, sl1_ref):
    d = pred_ref[...] - gt_ref[...]
    ad = jnp.abs(d)
    sl1 = jnp.where(ad < 1.0, 0.5 * d * d, ad - 0.5)
    s = jnp.sum(sl1, axis=1)
    masked = jnp.where(labels_ref[...] > 0, s, 0.0)
    sl1_ref[...] = jnp.sum(masked, axis=1, keepdims=True)


def _mine_body(neg_pos_ratio, n, bg_ref, ceps_ref, sl1s_ref, labels_ref,
               o1_ref, o2_ref):
    lab = labels_ref[...]
    pos = lab > 0
    num_pos = jnp.sum(pos.astype(jnp.int32), axis=1, keepdims=True)
    k = num_pos * neg_pos_ratio
    negcount = n - num_pos
    bg = bg_ref[...]
    bits = jax.lax.bitcast_convert_type(bg, jnp.uint32)
    key = jnp.where(bg >= 0, bits | jnp.uint32(0x80000000), ~bits)
    key = jnp.where(pos, jnp.uint32(0), key)
    sum_neg = jnp.sum(jnp.where(pos, 0.0, bg), axis=1, keepdims=True)
    prefix = jnp.zeros_like(num_pos, dtype=jnp.uint32)
    kk = k
    for bit in range(31, -1, -1):
        cand = prefix | jnp.uint32(1 << bit)
        match = (key >> jnp.uint32(bit)) == (cand >> jnp.uint32(bit))
        cnt = jnp.sum(match.astype(jnp.int32), axis=1, keepdims=True)
        take = cnt >= kk
        prefix = jnp.where(take, cand, prefix)
        kk = jnp.where(take, kk, kk - cnt)
    t = prefix
    gt_mask = key > t
    num_gt = jnp.sum(gt_mask.astype(jnp.int32), axis=1, keepdims=True)
    sum_gt = jnp.sum(jnp.where(gt_mask, bg, 0.0), axis=1, keepdims=True)
    vt = jnp.max(jnp.where(key == t, bg, -jnp.inf), axis=1, keepdims=True)
    topk = sum_gt + (k - num_gt).astype(jnp.float32) * vt
    topk = jnp.where(k >= negcount, sum_neg, topk)
    topk = jnp.where(k <= 0, 0.0, topk)
    npos_tot = jnp.sum(num_pos, axis=0, keepdims=True).astype(jnp.float32) + 1e-6
    sl1_tot = jnp.sum(sl1s_ref[...], axis=0, keepdims=True)
    cls_tot = jnp.sum(ceps_ref[...] + topk, axis=0, keepdims=True)
    o1_ref[...] = sl1_tot / npos_tot
    o2_ref[...] = cls_tot / npos_tot


def kernel(confidence, predicted_locations, labels, gt_locations):
    b, n, c = confidence.shape
    labels = labels.astype(jnp.int32)
    conf_t = confidence.transpose(2, 0, 1)
    pred_t = predicted_locations.transpose(0, 2, 1)
    gt_t = gt_locations.transpose(0, 2, 1)
    bb = 8
    cc = 9
    bg, ceps = pl.pallas_call(
        functools.partial(_lse_body, cc),
        grid=(b // bb, c // cc),
        in_specs=[
            pl.BlockSpec((cc, bb, n), lambda j, kc: (kc, j, 0)),
            pl.BlockSpec((bb, n), lambda j, kc: (j, 0)),
        ],
        out_specs=[
            pl.BlockSpec((bb, n), lambda j, kc: (j, 0)),
            pl.BlockSpec((bb, 1), lambda j, kc: (j, 0)),
        ],
        out_shape=[
            jax.ShapeDtypeStruct((b, n), jnp.float32),
            jax.ShapeDtypeStruct((b, 1), jnp.float32),
        ],
        scratch_shapes=[pltpu.VMEM((bb, n), jnp.float32) for _ in range(4)],
        interpret=_INTERPRET,
    )(conf_t, labels)

    sl1s = pl.pallas_call(
        _sl1_body,
        grid=(b // 8,),
        in_specs=[
            pl.BlockSpec((8, 4, n), lambda j: (j, 0, 0)),
            pl.BlockSpec((8, 4, n), lambda j: (j, 0, 0)),
            pl.BlockSpec((8, n), lambda j: (j, 0)),
        ],
        out_specs=pl.BlockSpec((8, 1), lambda j: (j, 0)),
        out_shape=jax.ShapeDtypeStruct((b, 1), jnp.float32),
        interpret=_INTERPRET,
    )(pred_t, gt_t, labels)

    o1, o2 = pl.pallas_call(
        functools.partial(_mine_body, _NEG_POS_RATIO, n),
        out_shape=[
            jax.ShapeDtypeStruct((1, 1), jnp.float32),
            jax.ShapeDtypeStruct((1, 1), jnp.float32),
        ],
        interpret=_INTERPRET,
    )(bg, ceps, sl1s, labels)
    return (o1[0, 0], o2[0, 0])


# speedup vs baseline: 1.3486x; 1.0914x over previous
import functools

import jax
import jax.numpy as jnp
from jax.experimental import pallas as pl
from jax.experimental.pallas import tpu as pltpu

_NEG_POS_RATIO = 3
_INTERPRET = False


def _lse_body(cc, conf_ref, labels_ref, bg_ref, cep_ref, m_s, s_s, xsel_s, x0_s):
    nc = pl.num_programs(1)
    kc = pl.program_id(1)
    lab = labels_ref[...]
    base = kc * cc
    mx = conf_ref[0]
    sel = jnp.where(lab == base, mx, 0.0)
    for ci in range(1, cc):
        xi = conf_ref[ci]
        mx = jnp.maximum(mx, xi)
        sel = sel + jnp.where(lab == base + ci, xi, 0.0)
    first = kc == 0
    m_old = jnp.where(first, mx, m_s[...])
    m_new = jnp.maximum(m_old, mx)
    s = jnp.where(first, 0.0, s_s[...]) * jnp.exp(m_old - m_new)
    for ci in range(cc):
        s = s + jnp.exp(conf_ref[ci] - m_new)
    m_s[...] = m_new
    s_s[...] = s
    xsel_s[...] = jnp.where(first, sel, xsel_s[...] + sel)

    @pl.when(first)
    def _save0():
        x0_s[...] = conf_ref[0]

    @pl.when(kc == nc - 1)
    def _emit():
        lse = m_new + jnp.log(s)
        bg_ref[...] = lse - x0_s[...]
        cep = jnp.where(lab > 0, lse - xsel_s[...], 0.0)
        cep_ref[...] = jnp.sum(cep, axis=1, keepdims=True)


def _sl1_body(pred_ref, gt_ref, labels_ref, sl1_ref):
    d = pred_ref[...] - gt_ref[...]
    ad = jnp.abs(d)
    sl1 = jnp.where(ad < 1.0, 0.5 * d * d, ad - 0.5)
    s = jnp.sum(sl1, axis=1)
    masked = jnp.where(labels_ref[...] > 0, s, 0.0)
    sl1_ref[...] = jnp.sum(masked, axis=1, keepdims=True)


def _mine_body(neg_pos_ratio, n, bg_ref, ceps_ref, sl1s_ref, labels_ref,
               o1_ref, o2_ref):
    lab = labels_ref[...]
    pos = lab > 0
    num_pos = jnp.sum(pos.astype(jnp.int32), axis=1, keepdims=True)
    k = num_pos * neg_pos_ratio
    negcount = n - num_pos
    bg = bg_ref[...]
    bits = jax.lax.bitcast_convert_type(bg, jnp.uint32)
    key = jnp.where(bg >= 0, bits | jnp.uint32(0x80000000), ~bits)
    key = jnp.where(pos, jnp.uint32(0), key)
    sum_neg = jnp.sum(jnp.where(pos, 0.0, bg), axis=1, keepdims=True)
    prefix = jnp.zeros_like(num_pos, dtype=jnp.uint32)
    kk = k
    for bit in range(31, -1, -1):
        cand = prefix | jnp.uint32(1 << bit)
        match = (key >> jnp.uint32(bit)) == (cand >> jnp.uint32(bit))
        cnt = jnp.sum(match.astype(jnp.int32), axis=1, keepdims=True)
        take = cnt >= kk
        prefix = jnp.where(take, cand, prefix)
        kk = jnp.where(take, kk, kk - cnt)
    t = prefix
    gt_mask = key > t
    num_gt = jnp.sum(gt_mask.astype(jnp.int32), axis=1, keepdims=True)
    sum_gt = jnp.sum(jnp.where(gt_mask, bg, 0.0), axis=1, keepdims=True)
    vt = jnp.max(jnp.where(key == t, bg, -jnp.inf), axis=1, keepdims=True)
    topk = sum_gt + (k - num_gt).astype(jnp.float32) * vt
    topk = jnp.where(k >= negcount, sum_neg, topk)
    topk = jnp.where(k <= 0, 0.0, topk)
    npos_tot = jnp.sum(num_pos, axis=0, keepdims=True).astype(jnp.float32) + 1e-6
    sl1_tot = jnp.sum(sl1s_ref[...], axis=0, keepdims=True)
    cls_tot = jnp.sum(ceps_ref[...] + topk, axis=0, keepdims=True)
    o1_ref[...] = sl1_tot / npos_tot
    o2_ref[...] = cls_tot / npos_tot


def kernel(confidence, predicted_locations, labels, gt_locations):
    b, n, c = confidence.shape
    labels = labels.astype(jnp.int32)
    conf_t = confidence.transpose(2, 0, 1)
    pred_t = predicted_locations.transpose(0, 2, 1)
    gt_t = gt_locations.transpose(0, 2, 1)
    bb = 8
    cc = 27
    bg, ceps = pl.pallas_call(
        functools.partial(_lse_body, cc),
        grid=(b // bb, c // cc),
        in_specs=[
            pl.BlockSpec((cc, bb, n), lambda j, kc: (kc, j, 0)),
            pl.BlockSpec((bb, n), lambda j, kc: (j, 0)),
        ],
        out_specs=[
            pl.BlockSpec((bb, n), lambda j, kc: (j, 0)),
            pl.BlockSpec((bb, 1), lambda j, kc: (j, 0)),
        ],
        out_shape=[
            jax.ShapeDtypeStruct((b, n), jnp.float32),
            jax.ShapeDtypeStruct((b, 1), jnp.float32),
        ],
        scratch_shapes=[pltpu.VMEM((bb, n), jnp.float32) for _ in range(4)],
        interpret=_INTERPRET,
    )(conf_t, labels)

    sl1s = pl.pallas_call(
        _sl1_body,
        grid=(b // 8,),
        in_specs=[
            pl.BlockSpec((8, 4, n), lambda j: (j, 0, 0)),
            pl.BlockSpec((8, 4, n), lambda j: (j, 0, 0)),
            pl.BlockSpec((8, n), lambda j: (j, 0)),
        ],
        out_specs=pl.BlockSpec((8, 1), lambda j: (j, 0)),
        out_shape=jax.ShapeDtypeStruct((b, 1), jnp.float32),
        interpret=_INTERPRET,
    )(pred_t, gt_t, labels)

    o1, o2 = pl.pallas_call(
        functools.partial(_mine_body, _NEG_POS_RATIO, n),
        out_shape=[
            jax.ShapeDtypeStruct((1, 1), jnp.float32),
            jax.ShapeDtypeStruct((1, 1), jnp.float32),
        ],
        interpret=_INTERPRET,
    )(bg, ceps, sl1s, labels)
    return (o1[0, 0], o2[0, 0])
